# baseline (speedup 1.0000x reference)
import jax
import jax.numpy as jnp
from jax.experimental import pallas as pl
from jax.experimental.pallas import tpu as pltpu

N = 1024
C = 5
Q = 15
NQ = C * Q
NH = N - NQ
D0, D1 = 256, 128
E0, E1 = 128, 128
FE = 64
KERN, STRIDE = 8, 4
CONV_OUT = (FE - KERN) // STRIDE + 1
HI = jax.lax.Precision.HIGHEST

BI = 512
NI = N // BI
JT = 128
NEG = -1e9


def _lrelu(x):
    return jnp.where(x > 0, x, 0.01 * x)


def _dgT(a, b):
    return jax.lax.dot_general(a, b, (((0,), (1,)), ((), ())), precision=HI)


def _colsum_T(p, y):
    return jax.lax.dot_general(p, y, (((0,), (0,)), ((), ())), precision=HI)


def _tl_col(labc):
    rows = jax.lax.broadcasted_iota(jnp.int32, (N, 1), 0)
    return jnp.where(rows >= NH, NH - 1 - rows, labc)


def _fused_kernel(labc, labr, f0, f1, W0, b0, W1, b1, Win, binr, Wl, Wr,
                  attc, bg, wconv, bconv, Wlab, blabr, out_o,
                  g_s, gl_s, gr_s, grT_s, agl_s, agr_s, z_s):
    step = pl.program_id(0)

    @pl.when(step == 0)
    def _pre():
        tlc = _tl_col(labc[...])
        classes = jax.lax.broadcasted_iota(jnp.int32, (1, C), 1)
        P = (tlc == classes).astype(jnp.float32)
        ones = jnp.ones((N, 1), jnp.float32)
        cnt = jnp.maximum(_colsum_T(P, ones), 1.0)
        rows = jax.lax.broadcasted_iota(jnp.int32, (N, 1), 0)
        qmask = (rows >= NH).astype(jnp.float32)

        def classmean(y):
            mean = _colsum_T(P, y) / cnt
            return jnp.dot(P, mean, precision=HI) + qmask * y

        y0 = jnp.dot(f0[...], W0[...], precision=HI)
        h0 = _lrelu(classmean(y0) + b0[...])
        y1 = jnp.dot(f1[...], W1[...], precision=HI)
        h1 = _lrelu(classmean(y1) + b1[...])
        oh = P + 0.2 * qmask

        Win_ = Win[...]
        u = (jnp.dot(h0, Win_[0:E0, :], precision=HI)
             + jnp.dot(h1, Win_[E0:E0 + E1, :], precision=HI)
             + jnp.dot(oh, Win_[E0 + E1:, :], precision=HI))
        g_s[...] = _lrelu(classmean(u) + binr[...])

        Wl_ = Wl[...]
        gl = (jnp.dot(h0, Wl_[0:E0, :], precision=HI)
              + jnp.dot(h1, Wl_[E0:E0 + E1, :], precision=HI)
              + jnp.dot(oh, Wl_[E0 + E1:, :], precision=HI))
        Wr_ = Wr[...]
        gr = (jnp.dot(h0, Wr_[0:E0, :], precision=HI)
              + jnp.dot(h1, Wr_[E0:E0 + E1, :], precision=HI)
              + jnp.dot(oh, Wr_[E0 + E1:, :], precision=HI))
        grT = (_dgT(Wr_[0:E0, :], h0) + _dgT(Wr_[E0:E0 + E1, :], h1)
               + _dgT(Wr_[E0 + E1:, :], oh))
        gl_s[...] = gl
        gr_s[...] = gr
        grT_s[...] = grT
        agl_s[...] = 0.6 * jnp.dot(gl, attc[...], precision=HI)
        agr_s[...] = 0.6 * _colsum_T(attc[...], grT)

    @pl.when((step >= 1) & (step <= NI))
    def _att():
        ib = step - 1
        row0 = pl.multiple_of(ib * BI, BI)
        glb = gl_s[pl.ds(row0, BI), :]
        grtb = grT_s[...]
        parts = []
        for jt in range(0, N, JT):
            acc = jnp.zeros((BI, JT), jnp.float32)
            for k in range(FE):
                s = glb[:, k:k + 1] + grtb[k:k + 1, jt:jt + JT]
                acc = acc + (0.4 * attc[k, 0]) * jnp.abs(s)
            parts.append(acc)
        e = (jnp.concatenate(parts, axis=1)
             + agl_s[pl.ds(row0, BI), :] + agr_s[...])
        ii = row0 + jax.lax.broadcasted_iota(jnp.int32, (BI, N), 0)
        jj = jax.lax.broadcasted_iota(jnp.int32, (BI, N), 1)
        rows_b = row0 + jax.lax.broadcasted_iota(jnp.int32, (BI, 1), 0)
        tlc_b = jnp.where(rows_b >= NH, NH - 1 - rows_b, labc[pl.ds(row0, BI), :])
        cols = jax.lax.broadcasted_iota(jnp.int32, (1, N), 1)
        tlr = jnp.where(cols >= NH, NH - 1 - cols, labr[...])
        allowed = (tlc_b != tlr) | (ii == jj)
        e = jnp.where(allowed, e, NEG)
        m = jnp.max(e, axis=1, keepdims=True)
        p = jnp.exp(e - m)
        alpha = p / jnp.sum(p, axis=1, keepdims=True)
        av = jnp.dot(alpha, gr_s[...], precision=HI) + bg[...]
        av = jnp.where(av > 0, av, jnp.exp(jnp.minimum(av, 0.0)) - 1.0)
        d = jax.lax.broadcasted_iota(jnp.int32, (FE, CONV_OUT), 0)
        t = jax.lax.broadcasted_iota(jnp.int32, (FE, CONV_OUT), 1)
        off = d - STRIDE * t
        Wc = jnp.zeros((FE, CONV_OUT), jnp.float32)
        for k in range(KERN):
            Wc = Wc + wconv[k, 0] * (off == k).astype(jnp.float32)
        aconv = jnp.dot(av, Wc, precision=HI) + bconv[0, 0]
        aconv = 1.0 / (1.0 + jnp.exp(-aconv))
        Wlab_ = Wlab[...]
        z_s[pl.ds(row0, BI), :] = (
            jnp.dot(g_s[pl.ds(row0, BI), :], Wlab_[0:FE, :], precision=HI)
            + jnp.dot(aconv, Wlab_[FE:, :], precision=HI))

    @pl.when(step == NI + 1)
    def _fin():
        classes = jax.lax.broadcasted_iota(jnp.int32, (1, C), 1)
        P = (labc[...] == classes).astype(jnp.float32)
        ones = jnp.ones((N, 1), jnp.float32)
        cnt = jnp.maximum(_colsum_T(P, ones), 1.0)
        mean = _colsum_T(P, z_s[...]) / cnt
        out_o[...] = jnp.dot(P, mean, precision=HI) + blabr[...]


def kernel(features_0, features_1, labels, W0, b0, W1, b1, Win, bin_,
           Wl, Wr, att, bg, wconv, bconv, Wlab, blab):
    labels = labels.astype(jnp.int32)
    f32 = jnp.float32
    shp = jax.ShapeDtypeStruct

    def cst(shape):
        return pl.BlockSpec(shape, lambda i: tuple(0 for _ in shape))

    vmem = pltpu.VMEM
    out = pl.pallas_call(
        _fused_kernel,
        grid=(NI + 2,),
        in_specs=[
            cst((N, 1)),
            cst((1, N)),
            cst((N, D0)), cst((N, D1)),
            cst((D0, E0)), cst((1, E0)),
            cst((D1, E1)), cst((1, E1)),
            cst((E0 + E1 + C, FE)), cst((1, FE)),
            cst((E0 + E1 + C, FE)),
            cst((E0 + E1 + C, FE)),
            cst((FE, 1)), cst((1, FE)),
            cst((KERN, 1)), cst((1, 1)),
            cst((FE + CONV_OUT, C)), cst((1, C)),
        ],
        out_specs=cst((N, C)),
        out_shape=shp((N, C), f32),
        scratch_shapes=[vmem((N, FE), f32), vmem((N, FE), f32),
                        vmem((N, FE), f32), vmem((FE, N), f32),
                        vmem((N, 1), f32), vmem((1, N), f32),
                        vmem((N, C), f32)],
    )(labels.reshape(N, 1), labels.reshape(1, N), features_0, features_1,
      W0, b0.reshape(1, E0), W1, b1.reshape(1, E1), Win,
      bin_.reshape(1, FE), Wl, Wr, att.reshape(FE, 1), bg.reshape(1, FE),
      wconv.reshape(KERN, 1), bconv.reshape(1, 1), Wlab, blab.reshape(1, C))
    return out


# speedup vs baseline: 1.2689x; 1.2689x over previous
import jax
import jax.numpy as jnp
from jax.experimental import pallas as pl
from jax.experimental.pallas import tpu as pltpu

N = 1024
C = 5
Q = 15
NQ = C * Q
NH = N - NQ
D0, D1 = 256, 128
E0, E1 = 128, 128
FE = 64
KERN, STRIDE = 8, 4
CONV_OUT = (FE - KERN) // STRIDE + 1
HI = jax.lax.Precision.HIGHEST

BI = 256
NI = N // BI
JT = 512
NEG = -1e9


def _lrelu(x):
    return jnp.where(x > 0, x, 0.01 * x)


def _dgT(a, b):
    return jax.lax.dot_general(a, b, (((0,), (1,)), ((), ())), precision=HI)


def _colsum_T(p, y):
    return jax.lax.dot_general(p, y, (((0,), (0,)), ((), ())), precision=HI)


def _tl_col(labc):
    rows = jax.lax.broadcasted_iota(jnp.int32, (N, 1), 0)
    return jnp.where(rows >= NH, NH - 1 - rows, labc)


def _fused_kernel(labc, labr, f0, f1, W0, b0, W1, b1, Win, binr, Wl, Wr,
                  attc, bg, wconv, bconv, Wlab, blabr, out_o,
                  g_s, gl_s, gr_s, grT_s, agl_s, agr_s, z_s):
    step = pl.program_id(0)

    @pl.when(step == 0)
    def _pre():
        tlc = _tl_col(labc[...])
        classes = jax.lax.broadcasted_iota(jnp.int32, (1, C), 1)
        P = (tlc == classes).astype(jnp.float32)
        ones = jnp.ones((N, 1), jnp.float32)
        cnt = jnp.maximum(_colsum_T(P, ones), 1.0)
        rows = jax.lax.broadcasted_iota(jnp.int32, (N, 1), 0)
        qmask = (rows >= NH).astype(jnp.float32)

        def classmean(y):
            mean = _colsum_T(P, y) / cnt
            return jnp.dot(P, mean, precision=HI) + qmask * y

        y0 = jnp.dot(f0[...], W0[...], precision=HI)
        h0 = _lrelu(classmean(y0) + b0[...])
        y1 = jnp.dot(f1[...], W1[...], precision=HI)
        h1 = _lrelu(classmean(y1) + b1[...])
        oh = P + 0.2 * qmask

        Win_ = Win[...]
        u = (jnp.dot(h0, Win_[0:E0, :], precision=HI)
             + jnp.dot(h1, Win_[E0:E0 + E1, :], precision=HI)
             + jnp.dot(oh, Win_[E0 + E1:, :], precision=HI))
        g_s[...] = _lrelu(classmean(u) + binr[...])

        Wl_ = Wl[...]
        gl = (jnp.dot(h0, Wl_[0:E0, :], precision=HI)
              + jnp.dot(h1, Wl_[E0:E0 + E1, :], precision=HI)
              + jnp.dot(oh, Wl_[E0 + E1:, :], precision=HI))
        Wr_ = Wr[...]
        gr = (jnp.dot(h0, Wr_[0:E0, :], precision=HI)
              + jnp.dot(h1, Wr_[E0:E0 + E1, :], precision=HI)
              + jnp.dot(oh, Wr_[E0 + E1:, :], precision=HI))
        grT = (_dgT(Wr_[0:E0, :], h0) + _dgT(Wr_[E0:E0 + E1, :], h1)
               + _dgT(Wr_[E0 + E1:, :], oh))
        gl_s[...] = gl
        gr_s[...] = gr
        grT_s[...] = grT
        agl_s[...] = 0.6 * jnp.dot(gl, attc[...], precision=HI)
        agr_s[...] = 0.6 * _colsum_T(attc[...], grT)

    @pl.when((step >= 1) & (step <= NI))
    def _att():
        ib = step - 1
        row0 = pl.multiple_of(ib * BI, BI)
        glb = gl_s[pl.ds(row0, BI), :]
        grtb = grT_s[...]
        parts = []
        for jt in range(0, N, JT):
            acc = jnp.zeros((BI, JT), jnp.float32)
            for k in range(FE):
                s = glb[:, k:k + 1] + grtb[k:k + 1, jt:jt + JT]
                acc = acc + (0.4 * attc[k, 0]) * jnp.abs(s)
            parts.append(acc)
        e = (jnp.concatenate(parts, axis=1)
             + agl_s[pl.ds(row0, BI), :] + agr_s[...])
        ii = row0 + jax.lax.broadcasted_iota(jnp.int32, (BI, N), 0)
        jj = jax.lax.broadcasted_iota(jnp.int32, (BI, N), 1)
        rows_b = row0 + jax.lax.broadcasted_iota(jnp.int32, (BI, 1), 0)
        tlc_b = jnp.where(rows_b >= NH, NH - 1 - rows_b, labc[pl.ds(row0, BI), :])
        cols = jax.lax.broadcasted_iota(jnp.int32, (1, N), 1)
        tlr = jnp.where(cols >= NH, NH - 1 - cols, labr[...])
        allowed = (tlc_b != tlr) | (ii == jj)
        e = jnp.where(allowed, e, NEG)
        m = jnp.max(e, axis=1, keepdims=True)
        p = jnp.exp(e - m)
        alpha = p / jnp.sum(p, axis=1, keepdims=True)
        av = jnp.dot(alpha, gr_s[...], precision=HI) + bg[...]
        av = jnp.where(av > 0, av, jnp.exp(jnp.minimum(av, 0.0)) - 1.0)
        d = jax.lax.broadcasted_iota(jnp.int32, (FE, CONV_OUT), 0)
        t = jax.lax.broadcasted_iota(jnp.int32, (FE, CONV_OUT), 1)
        off = d - STRIDE * t
        Wc = jnp.zeros((FE, CONV_OUT), jnp.float32)
        for k in range(KERN):
            Wc = Wc + wconv[k, 0] * (off == k).astype(jnp.float32)
        aconv = jnp.dot(av, Wc, precision=HI) + bconv[0, 0]
        aconv = 1.0 / (1.0 + jnp.exp(-aconv))
        Wlab_ = Wlab[...]
        z_s[pl.ds(row0, BI), :] = (
            jnp.dot(g_s[pl.ds(row0, BI), :], Wlab_[0:FE, :], precision=HI)
            + jnp.dot(aconv, Wlab_[FE:, :], precision=HI))

    @pl.when(step == NI + 1)
    def _fin():
        classes = jax.lax.broadcasted_iota(jnp.int32, (1, C), 1)
        P = (labc[...] == classes).astype(jnp.float32)
        ones = jnp.ones((N, 1), jnp.float32)
        cnt = jnp.maximum(_colsum_T(P, ones), 1.0)
        mean = _colsum_T(P, z_s[...]) / cnt
        out_o[...] = jnp.dot(P, mean, precision=HI) + blabr[...]


def kernel(features_0, features_1, labels, W0, b0, W1, b1, Win, bin_,
           Wl, Wr, att, bg, wconv, bconv, Wlab, blab):
    labels = labels.astype(jnp.int32)
    f32 = jnp.float32
    shp = jax.ShapeDtypeStruct

    def cst(shape):
        return pl.BlockSpec(shape, lambda i: tuple(0 for _ in shape))

    vmem = pltpu.VMEM
    out = pl.pallas_call(
        _fused_kernel,
        grid=(NI + 2,),
        in_specs=[
            cst((N, 1)),
            cst((1, N)),
            cst((N, D0)), cst((N, D1)),
            cst((D0, E0)), cst((1, E0)),
            cst((D1, E1)), cst((1, E1)),
            cst((E0 + E1 + C, FE)), cst((1, FE)),
            cst((E0 + E1 + C, FE)),
            cst((E0 + E1 + C, FE)),
            cst((FE, 1)), cst((1, FE)),
            cst((KERN, 1)), cst((1, 1)),
            cst((FE + CONV_OUT, C)), cst((1, C)),
        ],
        out_specs=cst((N, C)),
        out_shape=shp((N, C), f32),
        scratch_shapes=[vmem((N, FE), f32), vmem((N, FE), f32),
                        vmem((N, FE), f32), vmem((FE, N), f32),
                        vmem((N, 1), f32), vmem((1, N), f32),
                        vmem((N, C), f32)],
    )(labels.reshape(N, 1), labels.reshape(1, N), features_0, features_1,
      W0, b0.reshape(1, E0), W1, b1.reshape(1, E1), Win,
      bin_.reshape(1, FE), Wl, Wr, att.reshape(FE, 1), bg.reshape(1, FE),
      wconv.reshape(KERN, 1), bconv.reshape(1, 1), Wlab, blab.reshape(1, C))
    return out


# speedup vs baseline: 1.4176x; 1.1171x over previous
import jax
import jax.numpy as jnp
from jax.experimental import pallas as pl
from jax.experimental.pallas import tpu as pltpu

N = 1024
C = 5
Q = 15
NQ = C * Q
NH = N - NQ
D0, D1 = 256, 128
E0, E1 = 128, 128
FE = 64
KERN, STRIDE = 8, 4
CONV_OUT = (FE - KERN) // STRIDE + 1
HI = jax.lax.Precision.HIGHEST
DEF = jax.lax.Precision.DEFAULT

BI = 256
NI = N // BI
JT = 256
NEG = -1e9


def _lrelu(x):
    return jnp.where(x > 0, x, 0.01 * x)


def _dgT(a, b):
    return jax.lax.dot_general(a, b, (((0,), (1,)), ((), ())), precision=DEF)


def _colsum_T(p, y):
    return jax.lax.dot_general(p, y, (((0,), (0,)), ((), ())), precision=HI)


def _tl_col(labc):
    rows = jax.lax.broadcasted_iota(jnp.int32, (N, 1), 0)
    return jnp.where(rows >= NH, NH - 1 - rows, labc)


def _fused_kernel(labc, labr, f0, f1, W0, b0, W1, b1, Win, binr, Wl, Wr,
                  attc, bg, wconv, bconv, Wlab, blabr, out_o,
                  g_s, gl_s, gr_s, grT_s, agl_s, agr_s, z_s):
    step = pl.program_id(0)

    @pl.when(step == 0)
    def _pre():
        tlc = _tl_col(labc[...])
        classes = jax.lax.broadcasted_iota(jnp.int32, (1, C), 1)
        P = (tlc == classes).astype(jnp.float32)
        ones = jnp.ones((N, 1), jnp.float32)
        cnt = jnp.maximum(_colsum_T(P, ones), 1.0)
        rows = jax.lax.broadcasted_iota(jnp.int32, (N, 1), 0)
        qmask = (rows >= NH).astype(jnp.float32)

        def classmean(y):
            mean = _colsum_T(P, y) / cnt
            return jnp.dot(P, mean, precision=HI) + qmask * y

        y0 = jnp.dot(f0[...], W0[...], precision=DEF)
        h0 = _lrelu(classmean(y0) + b0[...])
        y1 = jnp.dot(f1[...], W1[...], precision=DEF)
        h1 = _lrelu(classmean(y1) + b1[...])
        oh = P + 0.2 * qmask

        Win_ = Win[...]
        u = (jnp.dot(h0, Win_[0:E0, :], precision=DEF)
             + jnp.dot(h1, Win_[E0:E0 + E1, :], precision=DEF)
             + jnp.dot(oh, Win_[E0 + E1:, :], precision=DEF))
        g_s[...] = _lrelu(classmean(u) + binr[...])

        Wl_ = Wl[...]
        gl = (jnp.dot(h0, Wl_[0:E0, :], precision=DEF)
              + jnp.dot(h1, Wl_[E0:E0 + E1, :], precision=DEF)
              + jnp.dot(oh, Wl_[E0 + E1:, :], precision=DEF))
        Wr_ = Wr[...]
        gr = (jnp.dot(h0, Wr_[0:E0, :], precision=DEF)
              + jnp.dot(h1, Wr_[E0:E0 + E1, :], precision=DEF)
              + jnp.dot(oh, Wr_[E0 + E1:, :], precision=DEF))
        grT = (_dgT(Wr_[0:E0, :], h0) + _dgT(Wr_[E0:E0 + E1, :], h1)
               + _dgT(Wr_[E0 + E1:, :], oh))
        gl_s[...] = gl
        gr_s[...] = gr
        grT_s[...] = grT
        agl_s[...] = 0.6 * jnp.dot(gl, attc[...], precision=HI)
        agr_s[...] = 0.6 * _colsum_T(attc[...], grT)

    @pl.when((step >= 1) & (step <= NI))
    def _att():
        ib = step - 1
        row0 = pl.multiple_of(ib * BI, BI)
        glb = gl_s[pl.ds(row0, BI), :]
        grtb = grT_s[...]
        parts = []
        for jt in range(0, N, JT):
            acc = jnp.zeros((BI, JT), jnp.float32)
            for k in range(FE):
                s = glb[:, k:k + 1] + grtb[k:k + 1, jt:jt + JT]
                acc = acc + (0.4 * attc[k, 0]) * jnp.abs(s)
            parts.append(acc)
        e = (jnp.concatenate(parts, axis=1)
             + agl_s[pl.ds(row0, BI), :] + agr_s[...])
        ii = row0 + jax.lax.broadcasted_iota(jnp.int32, (BI, N), 0)
        jj = jax.lax.broadcasted_iota(jnp.int32, (BI, N), 1)
        rows_b = row0 + jax.lax.broadcasted_iota(jnp.int32, (BI, 1), 0)
        tlc_b = jnp.where(rows_b >= NH, NH - 1 - rows_b, labc[pl.ds(row0, BI), :])
        cols = jax.lax.broadcasted_iota(jnp.int32, (1, N), 1)
        tlr = jnp.where(cols >= NH, NH - 1 - cols, labr[...])
        allowed = (tlc_b != tlr) | (ii == jj)
        e = jnp.where(allowed, e, NEG)
        m = jnp.max(e, axis=1, keepdims=True)
        p = jnp.exp(e - m)
        alpha = p / jnp.sum(p, axis=1, keepdims=True)
        av = jnp.dot(alpha, gr_s[...], precision=HI) + bg[...]
        av = jnp.where(av > 0, av, jnp.exp(jnp.minimum(av, 0.0)) - 1.0)
        d = jax.lax.broadcasted_iota(jnp.int32, (FE, CONV_OUT), 0)
        t = jax.lax.broadcasted_iota(jnp.int32, (FE, CONV_OUT), 1)
        off = d - STRIDE * t
        Wc = jnp.zeros((FE, CONV_OUT), jnp.float32)
        for k in range(KERN):
            Wc = Wc + wconv[k, 0] * (off == k).astype(jnp.float32)
        aconv = jnp.dot(av, Wc, precision=HI) + bconv[0, 0]
        aconv = 1.0 / (1.0 + jnp.exp(-aconv))
        Wlab_ = Wlab[...]
        z_s[pl.ds(row0, BI), :] = (
            jnp.dot(g_s[pl.ds(row0, BI), :], Wlab_[0:FE, :], precision=HI)
            + jnp.dot(aconv, Wlab_[FE:, :], precision=HI))

    @pl.when(step == NI + 1)
    def _fin():
        classes = jax.lax.broadcasted_iota(jnp.int32, (1, C), 1)
        P = (labc[...] == classes).astype(jnp.float32)
        ones = jnp.ones((N, 1), jnp.float32)
        cnt = jnp.maximum(_colsum_T(P, ones), 1.0)
        mean = _colsum_T(P, z_s[...]) / cnt
        out_o[...] = jnp.dot(P, mean, precision=HI) + blabr[...]


def kernel(features_0, features_1, labels, W0, b0, W1, b1, Win, bin_,
           Wl, Wr, att, bg, wconv, bconv, Wlab, blab):
    labels = labels.astype(jnp.int32)
    f32 = jnp.float32
    shp = jax.ShapeDtypeStruct

    def cst(shape):
        return pl.BlockSpec(shape, lambda i: tuple(0 for _ in shape))

    vmem = pltpu.VMEM
    out = pl.pallas_call(
        _fused_kernel,
        grid=(NI + 2,),
        in_specs=[
            cst((N, 1)),
            cst((1, N)),
            cst((N, D0)), cst((N, D1)),
            cst((D0, E0)), cst((1, E0)),
            cst((D1, E1)), cst((1, E1)),
            cst((E0 + E1 + C, FE)), cst((1, FE)),
            cst((E0 + E1 + C, FE)),
            cst((E0 + E1 + C, FE)),
            cst((FE, 1)), cst((1, FE)),
            cst((KERN, 1)), cst((1, 1)),
            cst((FE + CONV_OUT, C)), cst((1, C)),
        ],
        out_specs=cst((N, C)),
        out_shape=shp((N, C), f32),
        scratch_shapes=[vmem((N, FE), f32), vmem((N, FE), f32),
                        vmem((N, FE), f32), vmem((FE, N), f32),
                        vmem((N, 1), f32), vmem((1, N), f32),
                        vmem((N, C), f32)],
    )(labels.reshape(N, 1), labels.reshape(1, N), features_0, features_1,
      W0, b0.reshape(1, E0), W1, b1.reshape(1, E1), Win,
      bin_.reshape(1, FE), Wl, Wr, att.reshape(FE, 1), bg.reshape(1, FE),
      wconv.reshape(KERN, 1), bconv.reshape(1, 1), Wlab, blab.reshape(1, C))
    return out


# speedup vs baseline: 1.5956x; 1.1256x over previous
import jax
import jax.numpy as jnp
from jax.experimental import pallas as pl
from jax.experimental.pallas import tpu as pltpu

N = 1024
C = 5
Q = 15
NQ = C * Q
NH = N - NQ
D0, D1 = 256, 128
E0, E1 = 128, 128
FE = 64
KERN, STRIDE = 8, 4
CONV_OUT = (FE - KERN) // STRIDE + 1
HI = jax.lax.Precision.HIGHEST
DEF = jax.lax.Precision.DEFAULT

BI = 256
NI = N // BI
JT = 256
NEG = -1e9


def _lrelu(x):
    return jnp.where(x > 0, x, 0.01 * x)


def _dgT(a, b):
    return jax.lax.dot_general(a, b, (((0,), (1,)), ((), ())), precision=DEF)


def _colsum_T(p, y):
    return jax.lax.dot_general(p, y, (((0,), (0,)), ((), ())), precision=DEF)


def _tl_col(labc):
    rows = jax.lax.broadcasted_iota(jnp.int32, (N, 1), 0)
    return jnp.where(rows >= NH, NH - 1 - rows, labc)


def _fused_kernel(labc, labr, f0, f1, W0, b0, W1, b1, Win, binr, Wl, Wr,
                  attc, bg, wconv, bconv, Wlab, blabr, out_o,
                  g_s, gl_s, gr_s, grT_s, agl_s, agr_s, z_s):
    step = pl.program_id(0)

    @pl.when(step == 0)
    def _pre():
        tlc = _tl_col(labc[...])
        classes = jax.lax.broadcasted_iota(jnp.int32, (1, C), 1)
        P = (tlc == classes).astype(jnp.float32)
        ones = jnp.ones((N, 1), jnp.float32)
        cnt = jnp.maximum(_colsum_T(P, ones), 1.0)
        rows = jax.lax.broadcasted_iota(jnp.int32, (N, 1), 0)
        qmask = (rows >= NH).astype(jnp.float32)

        def classmean(y):
            mean = _colsum_T(P, y) / cnt
            return jnp.dot(P, mean, precision=DEF) + qmask * y

        y0 = jnp.dot(f0[...], W0[...], precision=DEF)
        h0 = _lrelu(classmean(y0) + b0[...])
        y1 = jnp.dot(f1[...], W1[...], precision=DEF)
        h1 = _lrelu(classmean(y1) + b1[...])
        oh = P + 0.2 * qmask

        Win_ = Win[...]
        u = (jnp.dot(h0, Win_[0:E0, :], precision=DEF)
             + jnp.dot(h1, Win_[E0:E0 + E1, :], precision=DEF)
             + jnp.dot(oh, Win_[E0 + E1:, :], precision=DEF))
        g_s[...] = _lrelu(classmean(u) + binr[...])

        Wl_ = Wl[...]
        gl = (jnp.dot(h0, Wl_[0:E0, :], precision=DEF)
              + jnp.dot(h1, Wl_[E0:E0 + E1, :], precision=DEF)
              + jnp.dot(oh, Wl_[E0 + E1:, :], precision=DEF))
        Wr_ = Wr[...]
        gr = (jnp.dot(h0, Wr_[0:E0, :], precision=DEF)
              + jnp.dot(h1, Wr_[E0:E0 + E1, :], precision=DEF)
              + jnp.dot(oh, Wr_[E0 + E1:, :], precision=DEF))
        grT = (_dgT(Wr_[0:E0, :], h0) + _dgT(Wr_[E0:E0 + E1, :], h1)
               + _dgT(Wr_[E0 + E1:, :], oh))
        gl_s[...] = gl
        gr_s[...] = gr
        grT_s[...] = grT
        agl_s[...] = 0.6 * jnp.dot(gl, attc[...], precision=DEF)
        agr_s[...] = 0.6 * _colsum_T(attc[...], grT)

    @pl.when((step >= 1) & (step <= NI))
    def _att():
        ib = step - 1
        row0 = pl.multiple_of(ib * BI, BI)
        glb = gl_s[pl.ds(row0, BI), :]
        grtb = grT_s[...]
        parts = []
        for jt in range(0, N, JT):
            acc = jnp.zeros((BI, JT), jnp.float32)
            for k in range(FE):
                s = glb[:, k:k + 1] + grtb[k:k + 1, jt:jt + JT]
                acc = acc + (0.4 * attc[k, 0]) * jnp.abs(s)
            parts.append(acc)
        e = (jnp.concatenate(parts, axis=1)
             + agl_s[pl.ds(row0, BI), :] + agr_s[...])
        ii = row0 + jax.lax.broadcasted_iota(jnp.int32, (BI, N), 0)
        jj = jax.lax.broadcasted_iota(jnp.int32, (BI, N), 1)
        rows_b = row0 + jax.lax.broadcasted_iota(jnp.int32, (BI, 1), 0)
        tlc_b = jnp.where(rows_b >= NH, NH - 1 - rows_b, labc[pl.ds(row0, BI), :])
        cols = jax.lax.broadcasted_iota(jnp.int32, (1, N), 1)
        tlr = jnp.where(cols >= NH, NH - 1 - cols, labr[...])
        allowed = (tlc_b != tlr) | (ii == jj)
        e = jnp.where(allowed, e, NEG)
        m = jnp.max(e, axis=1, keepdims=True)
        p = jnp.exp(e - m)
        alpha = p / jnp.sum(p, axis=1, keepdims=True)
        av = jnp.dot(alpha, gr_s[...], precision=DEF) + bg[...]
        av = jnp.where(av > 0, av, jnp.exp(jnp.minimum(av, 0.0)) - 1.0)
        d = jax.lax.broadcasted_iota(jnp.int32, (FE, CONV_OUT), 0)
        t = jax.lax.broadcasted_iota(jnp.int32, (FE, CONV_OUT), 1)
        off = d - STRIDE * t
        Wc = jnp.zeros((FE, CONV_OUT), jnp.float32)
        for k in range(KERN):
            Wc = Wc + wconv[k, 0] * (off == k).astype(jnp.float32)
        aconv = jnp.dot(av, Wc, precision=DEF) + bconv[0, 0]
        aconv = 1.0 / (1.0 + jnp.exp(-aconv))
        Wlab_ = Wlab[...]
        z_s[pl.ds(row0, BI), :] = (
            jnp.dot(g_s[pl.ds(row0, BI), :], Wlab_[0:FE, :], precision=DEF)
            + jnp.dot(aconv, Wlab_[FE:, :], precision=DEF))

    @pl.when(step == NI + 1)
    def _fin():
        classes = jax.lax.broadcasted_iota(jnp.int32, (1, C), 1)
        P = (labc[...] == classes).astype(jnp.float32)
        ones = jnp.ones((N, 1), jnp.float32)
        cnt = jnp.maximum(_colsum_T(P, ones), 1.0)
        mean = _colsum_T(P, z_s[...]) / cnt
        out_o[...] = jnp.dot(P, mean, precision=DEF) + blabr[...]


def kernel(features_0, features_1, labels, W0, b0, W1, b1, Win, bin_,
           Wl, Wr, att, bg, wconv, bconv, Wlab, blab):
    labels = labels.astype(jnp.int32)
    f32 = jnp.float32
    shp = jax.ShapeDtypeStruct

    def cst(shape):
        return pl.BlockSpec(shape, lambda i: tuple(0 for _ in shape))

    vmem = pltpu.VMEM
    out = pl.pallas_call(
        _fused_kernel,
        grid=(NI + 2,),
        in_specs=[
            cst((N, 1)),
            cst((1, N)),
            cst((N, D0)), cst((N, D1)),
            cst((D0, E0)), cst((1, E0)),
            cst((D1, E1)), cst((1, E1)),
            cst((E0 + E1 + C, FE)), cst((1, FE)),
            cst((E0 + E1 + C, FE)),
            cst((E0 + E1 + C, FE)),
            cst((FE, 1)), cst((1, FE)),
            cst((KERN, 1)), cst((1, 1)),
            cst((FE + CONV_OUT, C)), cst((1, C)),
        ],
        out_specs=cst((N, C)),
        out_shape=shp((N, C), f32),
        scratch_shapes=[vmem((N, FE), f32), vmem((N, FE), f32),
                        vmem((N, FE), f32), vmem((FE, N), f32),
                        vmem((N, 1), f32), vmem((1, N), f32),
                        vmem((N, C), f32)],
    )(labels.reshape(N, 1), labels.reshape(1, N), features_0, features_1,
      W0, b0.reshape(1, E0), W1, b1.reshape(1, E1), Win,
      bin_.reshape(1, FE), Wl, Wr, att.reshape(FE, 1), bg.reshape(1, FE),
      wconv.reshape(KERN, 1), bconv.reshape(1, 1), Wlab, blab.reshape(1, C))
    return out
